# baseline (speedup 1.0000x reference)
import functools

import jax
import jax.numpy as jnp
from jax import lax
from jax.experimental import pallas as pl
from jax.experimental.pallas import tpu as pltpu
from jax.experimental.pallas import tpu_sc as plsc

_N = 10000
_E = 320000
_L = 16
_NS = 16
_EPT = _E // _NS
_NPAD = 10240
_NPT = _NPAD // _NS


def _heat_body(col_hbm, ea_hbm, t_hbm, out_hbm,
               col_v, hist_v, red_v, deg_v, ea_v, out_v, t_v,
               sh_hist, sh_deg, sem_col, sem_ea):
    s = lax.axis_index("s")

    cp0 = pltpu.async_copy(col_hbm.at[pl.ds(s * _EPT, _EPT)], col_v, sem_col)
    cp1 = pltpu.async_copy(ea_hbm.at[pl.ds(s * _EPT, _EPT)], ea_v, sem_ea)

    zeros16 = jnp.zeros((_L,), jnp.float32)
    ones16 = jnp.ones((_L,), jnp.float32)

    def zero_body(j, carry):
        for u in range(8):
            hist_v[pl.ds((j * 8 + u) * _L, _L)] = zeros16
        return carry
    lax.fori_loop(0, _NPAD // _L // 8, zero_body, 0)

    cp0.wait()

    def hist_body(i, carry):
        for u in range(10):
            idx = col_v[pl.ds((i * 10 + u) * _L, _L)]
            plsc.addupdate_scatter(hist_v, [idx], ones16)
        return carry
    lax.fori_loop(0, _EPT // _L // 10, hist_body, 0)

    pltpu.sync_copy(hist_v, sh_hist.at[s])
    plsc.subcore_barrier()
    pltpu.sync_copy(sh_hist.at[:, pl.ds(s * _NPT, _NPT)], red_v)

    def red_body(j, carry):
        acc = red_v[0, pl.ds(j * _L, _L)]
        for r in range(1, _NS):
            acc = acc + red_v[r, pl.ds(j * _L, _L)]
        deg_v[pl.ds(j * _L, _L)] = acc
        return carry
    lax.fori_loop(0, _NPT // _L, red_body, 0)

    pltpu.sync_copy(deg_v.at[pl.ds(0, _NPT)], sh_deg.at[pl.ds(s * _NPT, _NPT)])
    plsc.subcore_barrier()
    pltpu.sync_copy(sh_deg, deg_v)

    cp1.wait()
    pltpu.sync_copy(t_hbm, t_v)
    tvec = t_v[...]

    def edge_body(i, carry):
        for u in range(5):
            k = i * 5 + u
            idx = col_v[pl.ds(k * _L, _L)]
            d = plsc.load_gather(deg_v, [idx])
            ea = ea_v[pl.ds(k * _L, _L)]
            out_v[pl.ds(k * _L, _L)] = jnp.exp(tvec * ea * (1.0 / d) - tvec)
        return carry
    lax.fori_loop(0, _EPT // _L // 5, edge_body, 0)

    pltpu.sync_copy(out_v, out_hbm.at[pl.ds(s * _EPT, _EPT)])


@jax.jit
def _heat_weights(col, edge_attr, tb):
    mesh = plsc.VectorSubcoreMesh(core_axis_name="c", subcore_axis_name="s",
                                  num_cores=1, num_subcores=_NS)
    return pl.kernel(
        _heat_body,
        out_type=jax.ShapeDtypeStruct((_E,), jnp.float32),
        mesh=mesh,
        compiler_params=pltpu.CompilerParams(needs_layout_passes=False),
        scratch_types=[
            pltpu.VMEM((_EPT,), jnp.int32),
            pltpu.VMEM((_NPAD,), jnp.float32),
            pltpu.VMEM((_NS, _NPT), jnp.float32),
            pltpu.VMEM((_NPAD,), jnp.float32),
            pltpu.VMEM((_EPT,), jnp.float32),
            pltpu.VMEM((_EPT,), jnp.float32),
            pltpu.VMEM((_L,), jnp.float32),
            pltpu.VMEM_SHARED((_NS, _NPAD), jnp.float32),
            pltpu.VMEM_SHARED((_NPAD,), jnp.float32),
            pltpu.SemaphoreType.DMA,
            pltpu.SemaphoreType.DMA,
        ],
    )(col, edge_attr, tb)


def kernel(x, edge_index, edge_attr, t):
    col = edge_index[1]
    tb = jnp.broadcast_to(t.astype(jnp.float32), (_L,))
    w = _heat_weights(col, edge_attr, tb)
    return (edge_index, w)


# speedup vs baseline: 1.1941x; 1.1941x over previous
import functools

import jax
import jax.numpy as jnp
from jax import lax
from jax.experimental import pallas as pl
from jax.experimental.pallas import tpu as pltpu
from jax.experimental.pallas import tpu_sc as plsc

_N = 10000
_E = 320000
_L = 16
_NC, _NS = 2, 16
_NW = _NC * _NS
_EPT = _E // _NW
_NPAD = 10240
_NPT = _NPAD // _NS
_C0 = 5040
_C1 = _EPT - _C0


def _heat_body(col_hbm, ea_hbm, t_hbm, out_hbm,
               col_v, hist_v, red_v, dinv_v, ea_v, out_v, t_v,
               sh_hist, sh_dinv, sem_col, sem_ea, sem_t, sem_deg, sem_out):
    c = lax.axis_index("c")
    s = lax.axis_index("s")
    wid = c * _NS + s

    cp0 = pltpu.async_copy(col_hbm.at[pl.ds(s * _EPT, _EPT)],
                           col_v.at[pl.ds(0, _EPT)], sem_col)
    cp1 = pltpu.async_copy(col_hbm.at[pl.ds((s + _NS) * _EPT, _EPT)],
                           col_v.at[pl.ds(_EPT, _EPT)], sem_col)
    cp2 = pltpu.async_copy(ea_hbm.at[pl.ds(wid * _EPT, _EPT)], ea_v, sem_ea)
    cp3 = pltpu.async_copy(t_hbm, t_v, sem_t)

    zeros16 = jnp.zeros((_L,), jnp.float32)
    ones16 = jnp.ones((_L,), jnp.float32)

    def zero_body(j, carry):
        for u in range(8):
            hist_v[pl.ds((j * 8 + u) * _L, _L)] = zeros16
        return carry
    lax.fori_loop(0, _NPAD // _L // 8, zero_body, 0)

    cp0.wait()

    def hist_body0(i, carry):
        for u in range(5):
            idx = col_v[pl.ds((i * 5 + u) * _L, _L)]
            plsc.addupdate_scatter(hist_v, [idx], ones16)
        return carry
    lax.fori_loop(0, _EPT // _L // 5, hist_body0, 0)

    cp1.wait()

    def hist_body1(i, carry):
        for u in range(5):
            idx = col_v[pl.ds(_EPT + (i * 5 + u) * _L, _L)]
            plsc.addupdate_scatter(hist_v, [idx], ones16)
        return carry
    lax.fori_loop(0, _EPT // _L // 5, hist_body1, 0)

    pltpu.sync_copy(hist_v, sh_hist.at[s])
    plsc.subcore_barrier()
    pltpu.sync_copy(sh_hist.at[:, pl.ds(s * _NPT, _NPT)], red_v)

    def red_body(j, carry):
        acc = red_v[0, pl.ds(j * _L, _L)]
        for r in range(1, _NS):
            acc = acc + red_v[r, pl.ds(j * _L, _L)]
        dinv_v[pl.ds(j * _L, _L)] = 1.0 / acc
        return carry
    lax.fori_loop(0, _NPT // _L, red_body, 0)

    pltpu.sync_copy(dinv_v.at[pl.ds(0, _NPT)], sh_dinv.at[pl.ds(s * _NPT, _NPT)])
    plsc.subcore_barrier()
    cp4 = pltpu.async_copy(sh_dinv, dinv_v, sem_deg)
    cp2.wait()
    cp3.wait()
    tvec = t_v[...]
    cp4.wait()

    base = c * _EPT

    def edge_body0(i, carry):
        for u in range(5):
            k = i * 5 + u
            idx = col_v[pl.ds(base + k * _L, _L)]
            dinv = plsc.load_gather(dinv_v, [idx])
            ea = ea_v[pl.ds(k * _L, _L)]
            out_v[pl.ds(k * _L, _L)] = jnp.exp(tvec * ea * dinv - tvec)
        return carry
    lax.fori_loop(0, _C0 // _L // 5, edge_body0, 0)
    cp5 = pltpu.async_copy(out_v.at[pl.ds(0, _C0)],
                           out_hbm.at[pl.ds(wid * _EPT, _C0)], sem_out)

    def edge_body1(i, carry):
        for u in range(5):
            k = _C0 // _L + i * 5 + u
            idx = col_v[pl.ds(base + k * _L, _L)]
            dinv = plsc.load_gather(dinv_v, [idx])
            ea = ea_v[pl.ds(k * _L, _L)]
            out_v[pl.ds(k * _L, _L)] = jnp.exp(tvec * ea * dinv - tvec)
        return carry
    lax.fori_loop(0, _C1 // _L // 5, edge_body1, 0)
    pltpu.sync_copy(out_v.at[pl.ds(_C0, _C1)],
                    out_hbm.at[pl.ds(wid * _EPT + _C0, _C1)])
    cp5.wait()


@jax.jit
def _heat_weights(col, edge_attr, tb):
    mesh = plsc.VectorSubcoreMesh(core_axis_name="c", subcore_axis_name="s",
                                  num_cores=_NC, num_subcores=_NS)
    return pl.kernel(
        _heat_body,
        out_type=jax.ShapeDtypeStruct((_E,), jnp.float32),
        mesh=mesh,
        compiler_params=pltpu.CompilerParams(needs_layout_passes=False),
        scratch_types=[
            pltpu.VMEM((2 * _EPT,), jnp.int32),
            pltpu.VMEM((_NPAD,), jnp.float32),
            pltpu.VMEM((_NS, _NPT), jnp.float32),
            pltpu.VMEM((_NPAD,), jnp.float32),
            pltpu.VMEM((_EPT,), jnp.float32),
            pltpu.VMEM((_EPT,), jnp.float32),
            pltpu.VMEM((_L,), jnp.float32),
            pltpu.VMEM_SHARED((_NS, _NPAD), jnp.float32),
            pltpu.VMEM_SHARED((_NPAD,), jnp.float32),
            pltpu.SemaphoreType.DMA,
            pltpu.SemaphoreType.DMA,
            pltpu.SemaphoreType.DMA,
            pltpu.SemaphoreType.DMA,
            pltpu.SemaphoreType.DMA,
        ],
    )(col, edge_attr, tb)


def kernel(x, edge_index, edge_attr, t):
    col = edge_index[1]
    tb = jnp.broadcast_to(t.astype(jnp.float32), (_L,))
    w = _heat_weights(col, edge_attr, tb)
    return (edge_index, w)
